# Initial kernel scaffold; baseline (speedup 1.0000x reference)
import jax
import jax.numpy as jnp
from jax.experimental import pallas as pl


def kernel(latent, codebook):
    raise NotImplementedError("write your pallas kernel here")


# speedup vs baseline: 3.2460x; 3.2460x over previous
import functools

import jax
import jax.numpy as jnp
from jax.experimental import pallas as pl
from jax.experimental.pallas import tpu as pltpu

CODEBOOK_SIZE = 128
EMBEDDING_DIM = 32
BATCH = 256
SEQ = 576

BATCH_PER_STEP = 8
TILE_ROWS = BATCH_PER_STEP * SEQ
GRID = BATCH // BATCH_PER_STEP


def _vq_body(lat_ref, cb_ref, policy_ref, quant_ref, cbset_ref):
    lat = lat_ref[...]
    cb = cb_ref[...]
    cb_norm = jnp.sum(cb * cb, axis=1)
    lat_norm = jnp.sum(lat * lat, axis=1, keepdims=True)
    scores = lat_norm + cb_norm[None, :] - 2.0 * jnp.dot(
        lat, cb.T, preferred_element_type=jnp.float32
    )
    smin = jnp.min(scores, axis=1, keepdims=True)
    code_iota = jax.lax.broadcasted_iota(
        jnp.int32, (TILE_ROWS, CODEBOOK_SIZE), 1
    )
    idx = jnp.min(
        jnp.where(scores == smin, code_iota, CODEBOOK_SIZE), axis=1
    )
    onehot = (idx[:, None] == code_iota).astype(jnp.float32)
    q = jnp.dot(onehot, cb, preferred_element_type=jnp.float32)
    quant_ref[...] = q
    policy_ref[...] = lat + (q - lat)
    cbset_ref[...] = jnp.broadcast_to(
        cb[None], (BATCH_PER_STEP, CODEBOOK_SIZE, EMBEDDING_DIM)
    )


@functools.partial(jax.jit, static_argnums=())
def kernel(latent, codebook):
    lat2d = latent.reshape(-1, EMBEDDING_DIM)
    policy, quant, cbset = pl.pallas_call(
        _vq_body,
        grid=(GRID,),
        in_specs=[
            pl.BlockSpec((TILE_ROWS, EMBEDDING_DIM), lambda i: (i, 0)),
            pl.BlockSpec((CODEBOOK_SIZE, EMBEDDING_DIM), lambda i: (0, 0)),
        ],
        out_specs=[
            pl.BlockSpec((TILE_ROWS, EMBEDDING_DIM), lambda i: (i, 0)),
            pl.BlockSpec((TILE_ROWS, EMBEDDING_DIM), lambda i: (i, 0)),
            pl.BlockSpec(
                (BATCH_PER_STEP, CODEBOOK_SIZE, EMBEDDING_DIM), lambda i: (i, 0, 0)
            ),
        ],
        out_shape=[
            jax.ShapeDtypeStruct((BATCH * SEQ, EMBEDDING_DIM), jnp.float32),
            jax.ShapeDtypeStruct((BATCH * SEQ, EMBEDDING_DIM), jnp.float32),
            jax.ShapeDtypeStruct((BATCH, CODEBOOK_SIZE, EMBEDDING_DIM), jnp.float32),
        ],
        compiler_params=pltpu.CompilerParams(
            dimension_semantics=("parallel",),
        ),
    )(lat2d, codebook)
    shape3 = (BATCH, SEQ, EMBEDDING_DIM)
    return policy.reshape(shape3), quant.reshape(shape3), cbset
